# baseline (speedup 1.0000x reference)
import functools

import jax
import jax.numpy as jnp
from jax import lax
from jax.experimental import pallas as pl
from jax.experimental.pallas import tpu as pltpu
from jax.experimental.pallas import tpu_sc as plsc

N = 2048
NPAIR = N // 2
L = 16
NGRP = NPAIR // L
NC, NS = 2, 16
NW = NC * NS
ROWS = 2 * 8192
RPW = ROWS // NW
R = 8
NCHUNK = RPW // R


@functools.partial(
    pl.kernel,
    out_type=jax.ShapeDtypeStruct((ROWS, N), jnp.float32),
    mesh=plsc.VectorSubcoreMesh(
        core_axis_name="c", subcore_axis_name="s", num_cores=NC, num_subcores=NS
    ),
    scratch_types=[
        pltpu.VMEM((NPAIR,), jnp.int32),
        pltpu.VMEM((NPAIR,), jnp.int32),
        pltpu.VMEM((NPAIR,), jnp.int32),
        pltpu.VMEM((NPAIR,), jnp.int32),
        pltpu.VMEM((NPAIR,), jnp.float32),
        pltpu.VMEM((NPAIR,), jnp.float32),
        pltpu.VMEM((R, N), jnp.float32),
        pltpu.VMEM((R, N), jnp.float32),
        pltpu.VMEM((R, N), jnp.float32),
        pltpu.VMEM((R, N), jnp.float32),
        pltpu.SemaphoreType.DMA,
        pltpu.SemaphoreType.DMA,
        pltpu.SemaphoreType.DMA,
        pltpu.SemaphoreType.DMA,
    ],
    compiler_params=pltpu.CompilerParams(
        use_tc_tiling_on_sc=True, needs_layout_passes=False
    ),
)
def _rot_sc(x_hbm, p0_hbm, p1_hbm, o0_hbm, o1_hbm, c_hbm, s_hbm, out_hbm,
            p0_v, p1_v, o0_v, o1_v, c_v, s_v,
            in0, in1, ou0, ou1, si0, si1, so0, so1):
    wid = lax.axis_index("s") * NC + lax.axis_index("c")
    base = wid * RPW
    pltpu.async_copy(p0_hbm, p0_v, so0)
    pltpu.async_copy(p1_hbm, p1_v, so0)
    pltpu.async_copy(o0_hbm, o0_v, so0)
    pltpu.async_copy(o1_hbm, o1_v, so0)
    pltpu.async_copy(c_hbm, c_v, so0)
    pltpu.async_copy(s_hbm, s_v, so0)

    ins, outs = (in0, in1), (ou0, ou1)
    sins, souts = (si0, si1), (so0, so1)

    def in_slice(ci):
        return x_hbm.at[pl.ds(base + ci * R, R)]

    def out_slice(ci):
        return out_hbm.at[pl.ds(base + ci * R, R)]

    pltpu.async_copy(in_slice(0), in0, si0)
    pltpu.async_copy(in_slice(1), in1, si1)
    pltpu.make_async_copy(p0_hbm, p0_v, so0).wait()
    pltpu.make_async_copy(p1_hbm, p1_v, so0).wait()
    pltpu.make_async_copy(o0_hbm, o0_v, so0).wait()
    pltpu.make_async_copy(o1_hbm, o1_v, so0).wait()
    pltpu.make_async_copy(c_hbm, c_v, so0).wait()
    pltpu.make_async_copy(s_hbm, s_v, so0).wait()

    def compute_chunk(b_in, b_out):
        @plsc.parallel_loop(0, NGRP, step=1, unroll=2)
        def grp_body(g):
            gb = g * L
            p0 = p0_v[pl.ds(gb, L)]
            p1 = p1_v[pl.ds(gb, L)]
            o0 = o0_v[pl.ds(gb, L)]
            o1 = o1_v[pl.ds(gb, L)]
            cc = c_v[pl.ds(gb, L)]
            ss = s_v[pl.ds(gb, L)]
            for r in range(R):
                rr = jnp.full((L,), r, jnp.int32)
                xi = plsc.load_gather(b_in, [rr, p0])
                xj = plsc.load_gather(b_in, [rr, p1])
                plsc.store_scatter(b_out, [rr, o0], cc * xi - ss * xj)
                plsc.store_scatter(b_out, [rr, o1], cc * xj + ss * xi)

    def pair_body(k, carry):
        for phase in range(2):
            ci = k * 2 + phase
            b_in, b_out = ins[phase], outs[phase]
            s_in, s_out = sins[phase], souts[phase]
            pltpu.make_async_copy(in_slice(ci), b_in, s_in).wait()

            @pl.when(k > 0)
            def _():
                pltpu.make_async_copy(b_out, out_slice(ci), s_out).wait()

            compute_chunk(b_in, b_out)
            pltpu.async_copy(b_out, out_slice(ci), s_out)

            @pl.when(ci + 2 < NCHUNK)
            def _():
                pltpu.async_copy(in_slice(ci + 2), b_in, s_in)
        return carry

    lax.fori_loop(0, NCHUNK // 2, pair_body, 0)
    pltpu.make_async_copy(ou0, out_slice(NCHUNK - 2), so0).wait()
    pltpu.make_async_copy(ou1, out_slice(NCHUNK - 1), so1).wait()


def kernel(inp, angles, pairs, outp_pairs):
    c = jnp.cos(angles)
    s = jnp.sin(angles)
    x = inp.reshape(ROWS, N)
    out = _rot_sc(
        x,
        pairs[:, 0], pairs[:, 1],
        outp_pairs[:, 0], outp_pairs[:, 1],
        c, s,
    )
    return out.reshape(inp.shape)


# speedup vs baseline: 1.3185x; 1.3185x over previous
import functools

import jax
import jax.numpy as jnp
from jax import lax
from jax.experimental import pallas as pl
from jax.experimental.pallas import tpu as pltpu
from jax.experimental.pallas import tpu_sc as plsc

N = 2048
NPAIR = N // 2
L = 16
NGRP = NPAIR // L
NC, NS = 2, 16
NW = NC * NS
ROWS = 2 * 8192
RPW = ROWS // NW
R = 8
NCHUNK = RPW // R


@functools.partial(
    pl.kernel,
    out_type=jax.ShapeDtypeStruct((ROWS, N), jnp.float32),
    mesh=plsc.VectorSubcoreMesh(
        core_axis_name="c", subcore_axis_name="s", num_cores=NC, num_subcores=NS
    ),
    scratch_types=[
        pltpu.VMEM((NPAIR,), jnp.int32),
        pltpu.VMEM((NPAIR,), jnp.int32),
        pltpu.VMEM((NPAIR,), jnp.int32),
        pltpu.VMEM((NPAIR,), jnp.int32),
        pltpu.VMEM((NPAIR,), jnp.float32),
        pltpu.VMEM((NPAIR,), jnp.float32),
        pltpu.VMEM((R, N), jnp.float32),
        pltpu.VMEM((R, N), jnp.float32),
        pltpu.VMEM((R, N), jnp.float32),
        pltpu.VMEM((R, N), jnp.float32),
        pltpu.SemaphoreType.DMA,
        pltpu.SemaphoreType.DMA,
        pltpu.SemaphoreType.DMA,
        pltpu.SemaphoreType.DMA,
    ],
    compiler_params=pltpu.CompilerParams(
        use_tc_tiling_on_sc=True, needs_layout_passes=False
    ),
)
def _rot_sc(x_hbm, p0_hbm, p1_hbm, o0_hbm, o1_hbm, c_hbm, s_hbm, out_hbm,
            p0_v, p1_v, o0_v, o1_v, c_v, s_v,
            in0, in1, ou0, ou1, si0, si1, so0, so1):
    wid = lax.axis_index("s") * NC + lax.axis_index("c")
    base = wid * RPW
    pltpu.async_copy(p0_hbm, p0_v, so0)
    pltpu.async_copy(p1_hbm, p1_v, so0)
    pltpu.async_copy(o0_hbm, o0_v, so0)
    pltpu.async_copy(o1_hbm, o1_v, so0)
    pltpu.async_copy(c_hbm, c_v, so0)
    pltpu.async_copy(s_hbm, s_v, so0)

    ins, outs = (in0, in1), (ou0, ou1)
    sins, souts = (si0, si1), (so0, so1)

    def in_slice(ci):
        return x_hbm.at[pl.ds(base + ci * R, R)]

    def out_slice(ci):
        return out_hbm.at[pl.ds(base + ci * R, R)]

    pltpu.async_copy(in_slice(0), in0, si0)
    pltpu.async_copy(in_slice(1), in1, si1)
    pltpu.make_async_copy(p0_hbm, p0_v, so0).wait()
    pltpu.make_async_copy(p1_hbm, p1_v, so0).wait()
    pltpu.make_async_copy(o0_hbm, o0_v, so0).wait()
    pltpu.make_async_copy(o1_hbm, o1_v, so0).wait()
    pltpu.make_async_copy(c_hbm, c_v, so0).wait()
    pltpu.make_async_copy(s_hbm, s_v, so0).wait()

    def compute_chunk(b_in, b_out):
        @plsc.parallel_loop(0, NGRP, step=1, unroll=2)
        def grp_body(g):
            gb = g * L
            p0 = p0_v[pl.ds(gb, L)]
            p1 = p1_v[pl.ds(gb, L)]
            o0 = o0_v[pl.ds(gb, L)]
            o1 = o1_v[pl.ds(gb, L)]
            cc = c_v[pl.ds(gb, L)]
            ss = s_v[pl.ds(gb, L)]
            xs = []
            for r in range(R):
                rr = jnp.full((L,), r, jnp.int32)
                xs.append((plsc.load_gather(b_in, [rr, p0]),
                           plsc.load_gather(b_in, [rr, p1])))
            for r in range(R):
                xi, xj = xs[r]
                rr = jnp.full((L,), r, jnp.int32)
                plsc.store_scatter(b_out, [rr, o0], cc * xi - ss * xj)
                plsc.store_scatter(b_out, [rr, o1], cc * xj + ss * xi)

    def pair_body(k, carry):
        for phase in range(2):
            ci = k * 2 + phase
            b_in, b_out = ins[phase], outs[phase]
            s_in, s_out = sins[phase], souts[phase]
            pltpu.make_async_copy(in_slice(ci), b_in, s_in).wait()

            @pl.when(k > 0)
            def _():
                pltpu.make_async_copy(b_out, out_slice(ci), s_out).wait()

            compute_chunk(b_in, b_out)
            pltpu.async_copy(b_out, out_slice(ci), s_out)

            @pl.when(ci + 2 < NCHUNK)
            def _():
                pltpu.async_copy(in_slice(ci + 2), b_in, s_in)
        return carry

    lax.fori_loop(0, NCHUNK // 2, pair_body, 0)
    pltpu.make_async_copy(ou0, out_slice(NCHUNK - 2), so0).wait()
    pltpu.make_async_copy(ou1, out_slice(NCHUNK - 1), so1).wait()


def kernel(inp, angles, pairs, outp_pairs):
    c = jnp.cos(angles)
    s = jnp.sin(angles)
    x = inp.reshape(ROWS, N)
    out = _rot_sc(
        x,
        pairs[:, 0], pairs[:, 1],
        outp_pairs[:, 0], outp_pairs[:, 1],
        c, s,
    )
    return out.reshape(inp.shape)
